# Initial kernel scaffold; baseline (speedup 1.0000x reference)
import functools

import jax
import jax.numpy as jnp
from jax import lax
from jax.experimental import pallas as pl
from jax.experimental.pallas import tpu as pltpu
from jax.experimental.pallas import tpu_sc as plsc

LANES = 16
CHUNK = 128
NC = 2
NS = 16
NW = NC * NS


def _proj_body(x_ref, w_ref, att2_ref, h_ref, a2_ref):
    h = jnp.dot(x_ref[...], w_ref[...], preferred_element_type=jnp.float32)
    h_ref[...] = h
    a2_ref[...] = jnp.dot(h, att2_ref[...], preferred_element_type=jnp.float32)


def _final_body(o_ref, d_ref, b_ref, out_ref):
    o = o_ref[0] + o_ref[1]
    d = d_ref[0] + d_ref[1]
    v = o / (d + 1e-16) + b_ref[...]
    out_ref[...] = jnp.where(v > 0, v, jnp.expm1(v))


def _make_sc_kernel(n_nodes, d_out, n_edges, ch_per_tile):
    per_tile = ch_per_tile * CHUNK
    n_per_tile = n_nodes // NS

    mesh = plsc.VectorSubcoreMesh(core_axis_name="c", subcore_axis_name="s")

    @functools.partial(
        pl.kernel,
        out_type=(
            jax.ShapeDtypeStruct((NC, n_nodes, d_out), jnp.float32),
            jax.ShapeDtypeStruct((NC, n_nodes), jnp.float32),
        ),
        mesh=mesh,
        scratch_types=[
            pltpu.VMEM((n_nodes,), jnp.float32),
            pltpu.VMEM((n_nodes,), jnp.float32),
            pltpu.VMEM((ch_per_tile, CHUNK), jnp.int32),
            pltpu.VMEM((ch_per_tile, CHUNK), jnp.int32),
            pltpu.VMEM((CHUNK,), jnp.float32),
            pltpu.VMEM((CHUNK, d_out), jnp.float32),
            pltpu.VMEM_SHARED((n_nodes, d_out), jnp.float32),
            pltpu.VMEM_SHARED((n_nodes,), jnp.float32),
            pltpu.SemaphoreType.DMA,
        ],
    )
    def sc_kernel(h_hbm, asrc_hbm, adst_hbm, src_hbm, dst_hbm,
                  outp_hbm, denp_hbm,
                  asrc_v, adst_v, src_v, dst_v, ex_v, rows_v,
                  out_sh, den_sh, sem):
        cid = lax.axis_index("c")
        sid = lax.axis_index("s")
        wid = cid * NS + sid

        pltpu.sync_copy(asrc_hbm, asrc_v)
        pltpu.sync_copy(adst_hbm, adst_v)
        pltpu.sync_copy(src_hbm.at[wid], src_v)
        pltpu.sync_copy(dst_hbm.at[wid], dst_v)

        def _zrow(r, _):
            for k in range(d_out // LANES):
                rows_v[r, pl.ds(k * LANES, LANES)] = jnp.zeros(
                    (LANES,), jnp.float32)
            return 0
        lax.fori_loop(0, CHUNK, _zrow, 0)
        for g in range(CHUNK // LANES):
            ex_v[pl.ds(g * LANES, LANES)] = jnp.zeros((LANES,), jnp.float32)

        off = 0
        while off < n_per_tile:
            blk = min(CHUNK, n_per_tile - off)
            pltpu.sync_copy(
                rows_v.at[pl.ds(0, blk)],
                out_sh.at[pl.ds(sid * n_per_tile + off, blk)])
            off += blk
        @pl.when(sid == 0)
        def _():
            off2 = 0
            while off2 < n_nodes:
                blk2 = min(CHUNK, n_nodes - off2)
                pltpu.sync_copy(ex_v.at[pl.ds(0, blk2)],
                                den_sh.at[pl.ds(off2, blk2)])
                off2 += blk2

        plsc.subcore_barrier()

        base_edge = wid * per_tile

        def chunk_body(c, _):
            src_row = src_v.at[c]
            dst_row = dst_v.at[c]
            gather = pltpu.async_copy(h_hbm.at[src_row], rows_v, sem)

            def g_body(g, _):
                s16 = src_v[c, pl.ds(g * LANES, LANES)]
                d16 = dst_v[c, pl.ds(g * LANES, LANES)]
                e = (plsc.load_gather(asrc_v, [s16])
                     + plsc.load_gather(adst_v, [d16]))
                e = jnp.where(e >= 0.0, e, 0.2 * e)
                ex = jnp.exp(e)
                gid = (base_edge + c * CHUNK + g * LANES
                       + lax.iota(jnp.int32, 16))
                ex = jnp.where(gid < n_edges, ex, 0.0)
                ex_v[pl.ds(g * LANES, LANES)] = ex
                return 0
            lax.fori_loop(0, CHUNK // LANES, g_body, 0)

            pltpu.sync_copy(ex_v, den_sh.at[dst_row], add=True)

            gather.wait()

            def m_body(g, _):
                for j in range(LANES):
                    r = g * LANES + j
                    sp = plsc.load_gather(
                        ex_v, [jnp.full((LANES,), r, jnp.int32)])
                    for k in range(d_out // LANES):
                        rows_v[r, pl.ds(k * LANES, LANES)] = (
                            rows_v[r, pl.ds(k * LANES, LANES)] * sp)
                return 0
            lax.fori_loop(0, CHUNK // LANES, m_body, 0)

            pltpu.sync_copy(rows_v, out_sh.at[dst_row], add=True)
            return 0

        lax.fori_loop(0, ch_per_tile, chunk_body, 0)

        plsc.subcore_barrier()

        pltpu.sync_copy(
            out_sh.at[pl.ds(sid * n_per_tile, n_per_tile)],
            outp_hbm.at[cid, pl.ds(sid * n_per_tile, n_per_tile)])

        @pl.when(sid == 0)
        def _():
            pltpu.sync_copy(den_sh, denp_hbm.at[cid])

    return sc_kernel


def kernel(input, edge_index, W, att_src, att_dst, bias):
    n, d_in = input.shape
    d_out = W.shape[1]
    n_edges = edge_index.shape[1]

    bn = 1000
    att2 = jnp.stack([att_src, att_dst], axis=1)
    h, a2 = pl.pallas_call(
        _proj_body,
        grid=(n // bn,),
        in_specs=[
            pl.BlockSpec((bn, d_in), lambda i: (i, 0)),
            pl.BlockSpec((d_in, d_out), lambda i: (0, 0)),
            pl.BlockSpec((d_out, 2), lambda i: (0, 0)),
        ],
        out_specs=[
            pl.BlockSpec((bn, d_out), lambda i: (i, 0)),
            pl.BlockSpec((bn, 2), lambda i: (i, 0)),
        ],
        out_shape=[
            jax.ShapeDtypeStruct((n, d_out), jnp.float32),
            jax.ShapeDtypeStruct((n, 2), jnp.float32),
        ],
    )(input, W, att2)
    asrc = a2[:, 0]
    adst = a2[:, 1]

    ch_per_tile = -(-n_edges // (NW * CHUNK))
    e_pad = NW * ch_per_tile * CHUNK
    src = edge_index[0].astype(jnp.int32)
    dst = edge_index[1].astype(jnp.int32)
    src = jnp.pad(src, (0, e_pad - n_edges)).reshape(NW, ch_per_tile, CHUNK)
    dst = jnp.pad(dst, (0, e_pad - n_edges)).reshape(NW, ch_per_tile, CHUNK)

    sc = _make_sc_kernel(n, d_out, n_edges, ch_per_tile)
    outp, denp = sc(h, asrc, adst, src, dst)

    denp3 = denp.reshape(NC, n, 1)
    bias2 = bias.reshape(1, d_out)
    out = pl.pallas_call(
        _final_body,
        grid=(n // bn,),
        in_specs=[
            pl.BlockSpec((NC, bn, d_out), lambda i: (0, i, 0)),
            pl.BlockSpec((NC, bn, 1), lambda i: (0, i, 0)),
            pl.BlockSpec((1, d_out), lambda i: (0, 0)),
        ],
        out_specs=pl.BlockSpec((bn, d_out), lambda i: (i, 0)),
        out_shape=jax.ShapeDtypeStruct((n, d_out), jnp.float32),
    )(outp, denp3, bias2)
    return out


# speedup vs baseline: 15.1121x; 15.1121x over previous
import functools

import jax
import jax.numpy as jnp
from jax import lax
from jax.experimental import pallas as pl
from jax.experimental.pallas import tpu as pltpu
from jax.experimental.pallas import tpu_sc as plsc

LANES = 16
CHUNK = 128
SUP = 8
NC = 2
NS = 16
NW = NC * NS


def _proj_body(x_ref, w_ref, att2_ref, h_ref, a2_ref):
    h = jnp.dot(x_ref[...], w_ref[...], preferred_element_type=jnp.float32)
    h_ref[...] = h
    a2_ref[...] = jnp.dot(h, att2_ref[...], preferred_element_type=jnp.float32)


def _final_body(o_ref, d_ref, b_ref, out_ref):
    o = o_ref[0] + o_ref[1]
    d = d_ref[0] + d_ref[1]
    v = o / (d + 1e-16) + b_ref[...]
    out_ref[...] = jnp.where(v > 0, v, jnp.exp(jnp.minimum(v, 0.0)) - 1.0)


def _make_sc_kernel(n_nodes, d_out, n_edges, sup_per_tile, n_pad):
    per_tile = sup_per_tile * SUP * CHUNK
    row_blk = (n_nodes // NS) // 8 * 8
    row_last = n_nodes - (NS - 1) * row_blk

    mesh = plsc.VectorSubcoreMesh(core_axis_name="c", subcore_axis_name="s")

    @functools.partial(
        pl.kernel,
        out_type=(
            jax.ShapeDtypeStruct((NC, n_nodes, d_out), jnp.float32),
            jax.ShapeDtypeStruct((NC * n_pad,), jnp.float32),
        ),
        mesh=mesh,
        compiler_params=pltpu.CompilerParams(needs_layout_passes=False),
        scratch_types=[
            pltpu.VMEM((SUP, CHUNK), jnp.int32),
            pltpu.VMEM((SUP, CHUNK), jnp.int32),
            pltpu.VMEM((CHUNK,), jnp.float32),
            pltpu.VMEM((CHUNK,), jnp.float32),
            pltpu.VMEM((CHUNK,), jnp.float32),
            pltpu.VMEM((CHUNK, d_out), jnp.float32),
            pltpu.VMEM_SHARED((n_nodes, d_out), jnp.float32),
            pltpu.VMEM_SHARED((n_pad,), jnp.float32),
            pltpu.SemaphoreType.DMA,
            pltpu.SemaphoreType.DMA,
            pltpu.SemaphoreType.DMA,
        ],
    )
    def sc_kernel(h_hbm, asrc_hbm, adst_hbm, src_hbm, dst_hbm,
                  outp_hbm, denp_hbm,
                  sidx_v, didx_v, av_v, bv_v, ex_v, rows_v,
                  out_sh, den_sh, semr, sema, semb):
        cid = lax.axis_index("c")
        sid = lax.axis_index("s")
        wid = cid * NS + sid

        def _zrow(r, _):
            for k in range(d_out // LANES):
                rows_v[r, pl.ds(k * LANES, LANES)] = jnp.zeros(
                    (LANES,), jnp.float32)
            return 0
        lax.fori_loop(0, CHUNK, _zrow, 0)
        for g in range(CHUNK // LANES):
            ex_v[pl.ds(g * LANES, LANES)] = jnp.zeros((LANES,), jnp.float32)

        def _zero_rows(start, count):
            off = 0
            while off < count:
                blk = min(CHUNK, count - off)
                pltpu.sync_copy(rows_v.at[pl.ds(0, blk)],
                                out_sh.at[pl.ds(start + off, blk)])
                off += blk

        @pl.when(sid < NS - 1)
        def _():
            _zero_rows(sid * row_blk, row_blk)

        @pl.when(sid == NS - 1)
        def _():
            _zero_rows((NS - 1) * row_blk, row_last)

        @pl.when(sid == 0)
        def _():
            for off2 in range(0, n_pad, CHUNK):
                pltpu.sync_copy(ex_v, den_sh.at[pl.ds(off2, CHUNK)])

        plsc.subcore_barrier()

        base_edge = wid * per_tile

        def super_body(cg, _):
            pltpu.sync_copy(src_hbm.at[wid, cg], sidx_v)
            pltpu.sync_copy(dst_hbm.at[wid, cg], didx_v)

            def chunk_body(r, _):
                src_row = sidx_v.at[r]
                dst_row = didx_v.at[r]
                rows_cp = pltpu.async_copy(h_hbm.at[src_row], rows_v, semr)
                a_cp = pltpu.async_copy(asrc_hbm.at[src_row], av_v, sema)
                b_cp = pltpu.async_copy(adst_hbm.at[dst_row], bv_v, semb)
                a_cp.wait()
                b_cp.wait()

                def g_body(g, _):
                    e = (av_v[pl.ds(g * LANES, LANES)]
                         + bv_v[pl.ds(g * LANES, LANES)])
                    e = jnp.where(e >= 0.0, e, 0.2 * e)
                    ex = jnp.exp(e)
                    gid = (base_edge + (cg * SUP + r) * CHUNK + g * LANES
                           + lax.iota(jnp.int32, 16))
                    ex = jnp.where(gid < n_edges, ex, 0.0)
                    ex_v[pl.ds(g * LANES, LANES)] = ex
                    return 0
                lax.fori_loop(0, CHUNK // LANES, g_body, 0)

                pltpu.sync_copy(ex_v, den_sh.at[dst_row], add=True)

                rows_cp.wait()

                def m_body(g, _):
                    for j in range(LANES):
                        rr = g * LANES + j
                        sp = plsc.load_gather(
                            ex_v, [jnp.full((LANES,), rr, jnp.int32)])
                        for k in range(d_out // LANES):
                            rows_v[rr, pl.ds(k * LANES, LANES)] = (
                                rows_v[rr, pl.ds(k * LANES, LANES)] * sp)
                    return 0
                lax.fori_loop(0, CHUNK // LANES, m_body, 0)

                pltpu.sync_copy(rows_v, out_sh.at[dst_row], add=True)
                return 0

            lax.fori_loop(0, SUP, chunk_body, 0)
            return 0

        lax.fori_loop(0, sup_per_tile, super_body, 0)

        plsc.subcore_barrier()

        @pl.when(sid < NS - 1)
        def _():
            pltpu.sync_copy(
                out_sh.at[pl.ds(sid * row_blk, row_blk)],
                outp_hbm.at[cid, pl.ds(sid * row_blk, row_blk)])

        @pl.when(sid == NS - 1)
        def _():
            pltpu.sync_copy(
                out_sh.at[pl.ds((NS - 1) * row_blk, row_last)],
                outp_hbm.at[cid, pl.ds((NS - 1) * row_blk, row_last)])

        @pl.when(sid == 0)
        def _():
            pltpu.sync_copy(den_sh, denp_hbm.at[pl.ds(cid * n_pad, n_pad)])

    return sc_kernel


def kernel(input, edge_index, W, att_src, att_dst, bias):
    n, d_in = input.shape
    d_out = W.shape[1]
    n_edges = edge_index.shape[1]

    bn = 1000
    att2 = jnp.stack([att_src, att_dst], axis=1)
    h, a2 = pl.pallas_call(
        _proj_body,
        grid=(n // bn,),
        in_specs=[
            pl.BlockSpec((bn, d_in), lambda i: (i, 0)),
            pl.BlockSpec((d_in, d_out), lambda i: (0, 0)),
            pl.BlockSpec((d_out, 2), lambda i: (0, 0)),
        ],
        out_specs=[
            pl.BlockSpec((bn, d_out), lambda i: (i, 0)),
            pl.BlockSpec((bn, 2), lambda i: (i, 0)),
        ],
        out_shape=[
            jax.ShapeDtypeStruct((n, d_out), jnp.float32),
            jax.ShapeDtypeStruct((n, 2), jnp.float32),
        ],
    )(input, W, att2)
    asrc = a2[:, 0]
    adst = a2[:, 1]

    sup_per_tile = -(-n_edges // (NW * SUP * CHUNK))
    e_pad = NW * sup_per_tile * SUP * CHUNK
    src = edge_index[0].astype(jnp.int32)
    dst = edge_index[1].astype(jnp.int32)
    src = jnp.pad(src, (0, e_pad - n_edges)).reshape(
        NW, sup_per_tile, SUP, CHUNK)
    dst = jnp.pad(dst, (0, e_pad - n_edges)).reshape(
        NW, sup_per_tile, SUP, CHUNK)

    n_pad = -(-n // 1024) * 1024
    sc = _make_sc_kernel(n, d_out, n_edges, sup_per_tile, n_pad)
    outp, denp = sc(h, asrc, adst, src, dst)

    denp3 = denp.reshape(NC, n_pad)[:, :n].reshape(NC, n, 1)
    bias2 = bias.reshape(1, d_out)
    out = pl.pallas_call(
        _final_body,
        grid=(n // bn,),
        in_specs=[
            pl.BlockSpec((NC, bn, d_out), lambda i: (0, i, 0)),
            pl.BlockSpec((NC, bn, 1), lambda i: (0, i, 0)),
            pl.BlockSpec((1, d_out), lambda i: (0, 0)),
        ],
        out_specs=pl.BlockSpec((bn, d_out), lambda i: (i, 0)),
        out_shape=jax.ShapeDtypeStruct((n, d_out), jnp.float32),
    )(outp, denp3, bias2)
    return out


# speedup vs baseline: 18.3184x; 1.2122x over previous
import functools

import jax
import jax.numpy as jnp
from jax import lax
from jax.experimental import pallas as pl
from jax.experimental.pallas import tpu as pltpu
from jax.experimental.pallas import tpu_sc as plsc

LANES = 16
CHUNK = 128
SUP = 16
NC = 2
NS = 16
NW = NC * NS


def _proj_body(x_ref, w_ref, att2_ref, h_ref, a2_ref):
    h = jnp.dot(x_ref[...], w_ref[...], preferred_element_type=jnp.float32)
    h_ref[...] = h
    a2_ref[...] = jnp.dot(h, att2_ref[...], preferred_element_type=jnp.float32)


def _final_body(o_ref, d_ref, b_ref, out_ref):
    o = o_ref[0] + o_ref[1]
    d = d_ref[0] + d_ref[1]
    v = o / (d + 1e-16) + b_ref[...]
    out_ref[...] = jnp.where(v > 0, v, jnp.exp(jnp.minimum(v, 0.0)) - 1.0)


def _make_sc_kernel(n_nodes, d_out, n_edges, sup_per_tile, n_pad):
    per_tile = sup_per_tile * SUP * CHUNK
    row_blk = (n_nodes // NS) // 8 * 8
    row_last = n_nodes - (NS - 1) * row_blk

    mesh = plsc.VectorSubcoreMesh(core_axis_name="c", subcore_axis_name="s")

    @functools.partial(
        pl.kernel,
        out_type=(
            jax.ShapeDtypeStruct((NC, n_nodes, d_out), jnp.float32),
            jax.ShapeDtypeStruct((NC * n_pad,), jnp.float32),
        ),
        mesh=mesh,
        compiler_params=pltpu.CompilerParams(needs_layout_passes=False),
        scratch_types=[
            pltpu.VMEM((2, SUP, CHUNK), jnp.int32),
            pltpu.VMEM((2, SUP, CHUNK), jnp.int32),
            pltpu.VMEM((2, CHUNK), jnp.float32),
            pltpu.VMEM((2, CHUNK), jnp.float32),
            pltpu.VMEM((2, CHUNK), jnp.float32),
            pltpu.VMEM((2, CHUNK, d_out), jnp.float32),
            pltpu.VMEM_SHARED((n_nodes, d_out), jnp.float32),
            pltpu.VMEM_SHARED((n_pad,), jnp.float32),
            pltpu.SemaphoreType.DMA,
            pltpu.SemaphoreType.DMA,
            pltpu.SemaphoreType.DMA,
            pltpu.SemaphoreType.DMA,
            pltpu.SemaphoreType.DMA,
            pltpu.SemaphoreType.DMA,
            pltpu.SemaphoreType.DMA,
            pltpu.SemaphoreType.DMA,
            pltpu.SemaphoreType.DMA,
            pltpu.SemaphoreType.DMA,
            pltpu.SemaphoreType.DMA,
        ],
    )
    def sc_kernel(h_hbm, asrc_hbm, adst_hbm, src_hbm, dst_hbm,
                  outp_hbm, denp_hbm,
                  sidx_v, didx_v, av_v, bv_v, ex_v, rows_v,
                  out_sh, den_sh, semidx,
                  semr0, semr1, sema0, sema1, semb0, semb1,
                  semd0, semd1, semo0, semo1):
        semr = (semr0, semr1)
        sema = (sema0, sema1)
        semb = (semb0, semb1)
        semd = (semd0, semd1)
        semo = (semo0, semo1)
        cid = lax.axis_index("c")
        sid = lax.axis_index("s")
        wid = cid * NS + sid

        def _zrow(r, _):
            for k in range(d_out // LANES):
                rows_v[0, r, pl.ds(k * LANES, LANES)] = jnp.zeros(
                    (LANES,), jnp.float32)
            return 0
        lax.fori_loop(0, CHUNK, _zrow, 0)
        for g in range(CHUNK // LANES):
            ex_v[0, pl.ds(g * LANES, LANES)] = jnp.zeros(
                (LANES,), jnp.float32)

        def _zero_rows(start, count):
            off = 0
            while off < count:
                blk = min(CHUNK, count - off)
                pltpu.sync_copy(rows_v.at[0, pl.ds(0, blk)],
                                out_sh.at[pl.ds(start + off, blk)])
                off += blk

        @pl.when(sid < NS - 1)
        def _():
            _zero_rows(sid * row_blk, row_blk)

        @pl.when(sid == NS - 1)
        def _():
            _zero_rows((NS - 1) * row_blk, row_last)

        @pl.when(sid == 0)
        def _():
            for off2 in range(0, n_pad, CHUNK):
                pltpu.sync_copy(ex_v.at[0], den_sh.at[pl.ds(off2, CHUNK)])

        plsc.subcore_barrier()

        base_edge = wid * per_tile
        n_blocks = sup_per_tile
        n_chunks = n_blocks * SUP

        def _idx_rows(t):
            q = (t // SUP) % 2
            r = t % SUP
            return sidx_v.at[q, r], didx_v.at[q, r]

        def _issue_gathers(t, p):
            s_row, d_row = _idx_rows(t)
            pltpu.async_copy(h_hbm.at[s_row], rows_v.at[p], semr[p])
            pltpu.async_copy(asrc_hbm.at[s_row], av_v.at[p], sema[p])
            pltpu.async_copy(adst_hbm.at[d_row], bv_v.at[p], semb[p])

        def _wait_scat(p):
            s_row, d_row = _idx_rows(0)
            pltpu.make_async_copy(ex_v.at[p], den_sh.at[d_row],
                                  semd[p]).wait()
            pltpu.make_async_copy(rows_v.at[p], out_sh.at[d_row],
                                  semo[p]).wait()

        def _process(t, p):
            s_row, d_row = _idx_rows(t)
            pltpu.make_async_copy(asrc_hbm.at[s_row], av_v.at[p],
                                  sema[p]).wait()
            pltpu.make_async_copy(adst_hbm.at[d_row], bv_v.at[p],
                                  semb[p]).wait()

            def g_body(g, _):
                e = (av_v[p, pl.ds(g * LANES, LANES)]
                     + bv_v[p, pl.ds(g * LANES, LANES)])
                e = jnp.where(e >= 0.0, e, 0.2 * e)
                ex = jnp.exp(e)
                gid = (base_edge + t * CHUNK + g * LANES
                       + lax.iota(jnp.int32, 16))
                ex = jnp.where(gid < n_edges, ex, 0.0)
                ex_v[p, pl.ds(g * LANES, LANES)] = ex
                return 0
            lax.fori_loop(0, CHUNK // LANES, g_body, 0)

            pltpu.async_copy(ex_v.at[p], den_sh.at[d_row], semd[p],
                             add=True)

            pltpu.make_async_copy(h_hbm.at[s_row], rows_v.at[p],
                                  semr[p]).wait()

            def m_body(g, _):
                for j in range(LANES):
                    rr = g * LANES + j
                    sp = plsc.load_gather(
                        ex_v.at[p], [jnp.full((LANES,), rr, jnp.int32)])
                    for k in range(d_out // LANES):
                        rows_v[p, rr, pl.ds(k * LANES, LANES)] = (
                            rows_v[p, rr, pl.ds(k * LANES, LANES)] * sp)
                return 0
            lax.fori_loop(0, CHUNK // LANES, m_body, 0)

            pltpu.async_copy(rows_v.at[p], out_sh.at[d_row], semo[p],
                             add=True)

        def _wait_idx_block(b):
            qb = b % 2
            pltpu.make_async_copy(src_hbm.at[wid, b], sidx_v.at[qb],
                                  semidx).wait()
            pltpu.make_async_copy(dst_hbm.at[wid, b], didx_v.at[qb],
                                  semidx).wait()

        pltpu.sync_copy(src_hbm.at[wid, 0], sidx_v.at[0])
        pltpu.sync_copy(dst_hbm.at[wid, 0], didx_v.at[0])
        _issue_gathers(0, 0)

        def pipe_body(u, _):
            ta = 2 * u
            tb = ta + 1
            @pl.when(ta % SUP == 0)
            def _():
                b = ta // SUP

                @pl.when(b + 1 < n_blocks)
                def _():
                    qn = (b + 1) % 2
                    pltpu.async_copy(src_hbm.at[wid, b + 1], sidx_v.at[qn],
                                     semidx)
                    pltpu.async_copy(dst_hbm.at[wid, b + 1], didx_v.at[qn],
                                     semidx)

            @pl.when(u > 0)
            def _():
                _wait_scat(1)
            _issue_gathers(tb, 1)

            _process(ta, 0)
            _process(tb, 1)

            tn = ta + 2

            @pl.when(tn < n_chunks)
            def _():
                @pl.when(tn % SUP == 0)
                def _():
                    _wait_idx_block(tn // SUP)
                _wait_scat(0)
                _issue_gathers(tn, 0)
            return 0

        lax.fori_loop(0, n_chunks // 2, pipe_body, 0)

        _wait_scat(0)
        _wait_scat(1)

        plsc.subcore_barrier()

        @pl.when(sid < NS - 1)
        def _():
            pltpu.sync_copy(
                out_sh.at[pl.ds(sid * row_blk, row_blk)],
                outp_hbm.at[cid, pl.ds(sid * row_blk, row_blk)])

        @pl.when(sid == NS - 1)
        def _():
            pltpu.sync_copy(
                out_sh.at[pl.ds((NS - 1) * row_blk, row_last)],
                outp_hbm.at[cid, pl.ds((NS - 1) * row_blk, row_last)])

        @pl.when(sid == 0)
        def _():
            pltpu.sync_copy(den_sh, denp_hbm.at[pl.ds(cid * n_pad, n_pad)])

    return sc_kernel


def kernel(input, edge_index, W, att_src, att_dst, bias):
    n, d_in = input.shape
    d_out = W.shape[1]
    n_edges = edge_index.shape[1]

    bn = 1000
    att2 = jnp.stack([att_src, att_dst], axis=1)
    h, a2 = pl.pallas_call(
        _proj_body,
        grid=(n // bn,),
        in_specs=[
            pl.BlockSpec((bn, d_in), lambda i: (i, 0)),
            pl.BlockSpec((d_in, d_out), lambda i: (0, 0)),
            pl.BlockSpec((d_out, 2), lambda i: (0, 0)),
        ],
        out_specs=[
            pl.BlockSpec((bn, d_out), lambda i: (i, 0)),
            pl.BlockSpec((bn, 2), lambda i: (i, 0)),
        ],
        out_shape=[
            jax.ShapeDtypeStruct((n, d_out), jnp.float32),
            jax.ShapeDtypeStruct((n, 2), jnp.float32),
        ],
    )(input, W, att2)
    asrc = a2[:, 0]
    adst = a2[:, 1]

    sup_per_tile = -(-n_edges // (NW * SUP * CHUNK))
    e_pad = NW * sup_per_tile * SUP * CHUNK
    src = edge_index[0].astype(jnp.int32)
    dst = edge_index[1].astype(jnp.int32)
    src = jnp.pad(src, (0, e_pad - n_edges)).reshape(
        NW, sup_per_tile, SUP, CHUNK)
    dst = jnp.pad(dst, (0, e_pad - n_edges)).reshape(
        NW, sup_per_tile, SUP, CHUNK)

    n_pad = -(-n // 1024) * 1024
    sc = _make_sc_kernel(n, d_out, n_edges, sup_per_tile, n_pad)
    outp, denp = sc(h, asrc, adst, src, dst)

    denp3 = denp.reshape(NC, n_pad)[:, :n].reshape(NC, n, 1)
    bias2 = bias.reshape(1, d_out)
    out = pl.pallas_call(
        _final_body,
        grid=(n // bn,),
        in_specs=[
            pl.BlockSpec((NC, bn, d_out), lambda i: (0, i, 0)),
            pl.BlockSpec((NC, bn, 1), lambda i: (0, i, 0)),
            pl.BlockSpec((1, d_out), lambda i: (0, 0)),
        ],
        out_specs=pl.BlockSpec((bn, d_out), lambda i: (i, 0)),
        out_shape=jax.ShapeDtypeStruct((n, d_out), jnp.float32),
    )(outp, denp3, bias2)
    return out


# speedup vs baseline: 43.1078x; 2.3533x over previous
import functools

import jax
import jax.numpy as jnp
from jax import lax
from jax.experimental import pallas as pl
from jax.experimental.pallas import tpu as pltpu
from jax.experimental.pallas import tpu_sc as plsc

LANES = 16
CHUNK = 128
SUP = 16
NC = 2
NS = 16
NW = NC * NS


def _proj_body(x_ref, w_ref, att2_ref, h_ref, a2_ref):
    h = jnp.dot(x_ref[...], w_ref[...], preferred_element_type=jnp.float32)
    h_ref[...] = h
    a2_ref[...] = jnp.dot(h, att2_ref[...], preferred_element_type=jnp.float32)


def _final_body(o_ref, d_ref, b_ref, out_ref):
    o = o_ref[0] + o_ref[1]
    d = d_ref[0] + d_ref[1]
    v = o / (d + 1e-16) + b_ref[...]
    out_ref[...] = jnp.where(v > 0, v, jnp.exp(jnp.minimum(v, 0.0)) - 1.0)


def _make_sc_kernel(n_nodes, d_out, n_edges, sup_per_tile, n_pad):
    per_tile = sup_per_tile * SUP * CHUNK
    row_blk = (n_nodes // NS) // 8 * 8
    row_last = n_nodes - (NS - 1) * row_blk

    mesh = plsc.VectorSubcoreMesh(core_axis_name="c", subcore_axis_name="s")

    @functools.partial(
        pl.kernel,
        out_type=(
            jax.ShapeDtypeStruct((NC, n_nodes, d_out), jnp.float32),
            jax.ShapeDtypeStruct((NC * n_pad,), jnp.float32),
        ),
        mesh=mesh,
        compiler_params=pltpu.CompilerParams(needs_layout_passes=False),
        scratch_types=[
            pltpu.VMEM((2, SUP, CHUNK), jnp.int32),
            pltpu.VMEM((2, SUP, CHUNK), jnp.int32),
            pltpu.VMEM((2, CHUNK), jnp.float32),
            pltpu.VMEM((2, CHUNK), jnp.float32),
            pltpu.VMEM((2, CHUNK), jnp.float32),
            pltpu.VMEM((2, CHUNK, d_out), jnp.float32),
            pltpu.VMEM_SHARED((n_nodes + CHUNK, d_out), jnp.float32),
            pltpu.VMEM_SHARED((n_pad,), jnp.float32),
            pltpu.SemaphoreType.DMA,
            pltpu.SemaphoreType.DMA,
            pltpu.SemaphoreType.DMA,
            pltpu.SemaphoreType.DMA,
            pltpu.SemaphoreType.DMA,
            pltpu.SemaphoreType.DMA,
            pltpu.SemaphoreType.DMA,
            pltpu.SemaphoreType.DMA,
            pltpu.SemaphoreType.DMA,
            pltpu.SemaphoreType.DMA,
            pltpu.SemaphoreType.DMA,
        ],
    )
    def sc_kernel(h_hbm, asrc_hbm, adst_hbm, src_hbm, dst_hbm,
                  outp_hbm, denp_hbm,
                  sidx_v, didx_v, av_v, bv_v, ex_v, rows_v,
                  out_sh, den_sh, semidx,
                  semr0, semr1, sema0, sema1, semb0, semb1,
                  semd0, semd1, semo0, semo1):
        semr = (semr0, semr1)
        sema = (sema0, sema1)
        semb = (semb0, semb1)
        semd = (semd0, semd1)
        semo = (semo0, semo1)
        cid = lax.axis_index("c")
        sid = lax.axis_index("s")
        wid = cid * NS + sid

        def _zrow(r, _):
            for k in range(d_out // LANES):
                rows_v[0, r, pl.ds(k * LANES, LANES)] = jnp.zeros(
                    (LANES,), jnp.float32)
            return 0
        lax.fori_loop(0, CHUNK, _zrow, 0)
        for g in range(CHUNK // LANES):
            ex_v[0, pl.ds(g * LANES, LANES)] = jnp.zeros(
                (LANES,), jnp.float32)

        def _zero_rows(start, count):
            off = 0
            while off < count:
                blk = min(CHUNK, count - off)
                pltpu.sync_copy(rows_v.at[0, pl.ds(0, blk)],
                                out_sh.at[pl.ds(start + off, blk)])
                off += blk

        @pl.when(sid < NS - 1)
        def _():
            _zero_rows(sid * row_blk, row_blk)

        @pl.when(sid == NS - 1)
        def _():
            _zero_rows((NS - 1) * row_blk, row_last)

        @pl.when(sid == 0)
        def _():
            for off2 in range(0, n_pad, CHUNK):
                pltpu.sync_copy(ex_v.at[0], den_sh.at[pl.ds(off2, CHUNK)])

        plsc.subcore_barrier()

        base_edge = wid * per_tile
        n_blocks = sup_per_tile
        n_chunks = n_blocks * SUP

        def _idx_rows(t):
            q = (t // SUP) % 2
            r = t % SUP
            return sidx_v.at[q, r], didx_v.at[q, r]

        def _issue_gathers(t, p):
            s_row, d_row = _idx_rows(t)
            pltpu.async_copy(h_hbm.at[s_row], rows_v.at[p], semr[p])
            pltpu.async_copy(asrc_hbm.at[s_row], av_v.at[p], sema[p])
            pltpu.async_copy(adst_hbm.at[d_row], bv_v.at[p], semb[p])

        def _wait_scat(p):
            s_row, d_row = _idx_rows(0)
            pltpu.make_async_copy(ex_v.at[p], den_sh.at[d_row],
                                  semd[p]).wait()
            pltpu.make_async_copy(rows_v.at[p], out_sh.at[d_row],
                                  semo[p]).wait()

        def _process(t, p):
            s_row, d_row = _idx_rows(t)
            pltpu.make_async_copy(asrc_hbm.at[s_row], av_v.at[p],
                                  sema[p]).wait()
            pltpu.make_async_copy(adst_hbm.at[d_row], bv_v.at[p],
                                  semb[p]).wait()

            def g_body(g, _):
                e = (av_v[p, pl.ds(g * LANES, LANES)]
                     + bv_v[p, pl.ds(g * LANES, LANES)])
                e = jnp.where(e >= 0.0, e, 0.2 * e)
                ex = jnp.exp(e)
                gid = (base_edge + t * CHUNK + g * LANES
                       + lax.iota(jnp.int32, 16))
                ex = jnp.where(gid < n_edges, ex, 0.0)
                ex_v[p, pl.ds(g * LANES, LANES)] = ex
                return 0
            lax.fori_loop(0, CHUNK // LANES, g_body, 0)

            pltpu.async_copy(ex_v.at[p], den_sh.at[d_row], semd[p],
                             add=True)

            pltpu.make_async_copy(h_hbm.at[s_row], rows_v.at[p],
                                  semr[p]).wait()

            def m_body(g, _):
                for j in range(LANES):
                    rr = g * LANES + j
                    sp = plsc.load_gather(
                        ex_v.at[p], [jnp.full((LANES,), rr, jnp.int32)])
                    for k in range(d_out // LANES):
                        rows_v[p, rr, pl.ds(k * LANES, LANES)] = (
                            rows_v[p, rr, pl.ds(k * LANES, LANES)] * sp)
                return 0
            lax.fori_loop(0, CHUNK // LANES, m_body, 0)

            pltpu.async_copy(rows_v.at[p], out_sh.at[d_row], semo[p],
                             add=True)

        def _wait_idx_block(b):
            qb = b % 2
            pltpu.make_async_copy(src_hbm.at[wid, b], sidx_v.at[qb],
                                  semidx).wait()
            pltpu.make_async_copy(dst_hbm.at[wid, b], didx_v.at[qb],
                                  semidx).wait()

        pltpu.sync_copy(src_hbm.at[wid, 0], sidx_v.at[0])
        pltpu.sync_copy(dst_hbm.at[wid, 0], didx_v.at[0])
        _issue_gathers(0, 0)

        def pipe_body(u, _):
            ta = 2 * u
            tb = ta + 1
            @pl.when(ta % SUP == 0)
            def _():
                b = ta // SUP

                @pl.when(b + 1 < n_blocks)
                def _():
                    qn = (b + 1) % 2
                    pltpu.async_copy(src_hbm.at[wid, b + 1], sidx_v.at[qn],
                                     semidx)
                    pltpu.async_copy(dst_hbm.at[wid, b + 1], didx_v.at[qn],
                                     semidx)

            @pl.when(u > 0)
            def _():
                _wait_scat(1)
            _issue_gathers(tb, 1)

            _process(ta, 0)
            _process(tb, 1)

            tn = ta + 2

            @pl.when(tn < n_chunks)
            def _():
                @pl.when(tn % SUP == 0)
                def _():
                    _wait_idx_block(tn // SUP)
                _wait_scat(0)
                _issue_gathers(tn, 0)
            return 0

        lax.fori_loop(0, n_chunks // 2, pipe_body, 0)

        _wait_scat(0)
        _wait_scat(1)

        plsc.subcore_barrier()

        @pl.when(sid < NS - 1)
        def _():
            pltpu.sync_copy(
                out_sh.at[pl.ds(sid * row_blk, row_blk)],
                outp_hbm.at[cid, pl.ds(sid * row_blk, row_blk)])

        @pl.when(sid == NS - 1)
        def _():
            pltpu.sync_copy(
                out_sh.at[pl.ds((NS - 1) * row_blk, row_last)],
                outp_hbm.at[cid, pl.ds((NS - 1) * row_blk, row_last)])

        @pl.when(sid == 0)
        def _():
            pltpu.sync_copy(den_sh, denp_hbm.at[pl.ds(cid * n_pad, n_pad)])

    return sc_kernel


def kernel(input, edge_index, W, att_src, att_dst, bias):
    n, d_in = input.shape
    d_out = W.shape[1]
    n_edges = edge_index.shape[1]

    bn = 1000
    att2 = jnp.stack([att_src, att_dst], axis=1)
    h, a2 = pl.pallas_call(
        _proj_body,
        grid=(n // bn,),
        in_specs=[
            pl.BlockSpec((bn, d_in), lambda i: (i, 0)),
            pl.BlockSpec((d_in, d_out), lambda i: (0, 0)),
            pl.BlockSpec((d_out, 2), lambda i: (0, 0)),
        ],
        out_specs=[
            pl.BlockSpec((bn, d_out), lambda i: (i, 0)),
            pl.BlockSpec((bn, 2), lambda i: (i, 0)),
        ],
        out_shape=[
            jax.ShapeDtypeStruct((n, d_out), jnp.float32),
            jax.ShapeDtypeStruct((n, 2), jnp.float32),
        ],
    )(input, W, att2)
    asrc = a2[:, 0]
    adst = a2[:, 1]

    sup_per_tile = -(-n_edges // (NW * SUP * CHUNK))
    e_pad = NW * sup_per_tile * SUP * CHUNK
    src = edge_index[0].astype(jnp.int32)
    dst = edge_index[1].astype(jnp.int32)
    npad_e = e_pad - n_edges
    pad_iota = lax.iota(jnp.int32, npad_e)
    src = jnp.concatenate([src, pad_iota % n]).reshape(
        NW, sup_per_tile, SUP, CHUNK)
    dst = jnp.concatenate([dst, n + (pad_iota % CHUNK)]).reshape(
        NW, sup_per_tile, SUP, CHUNK)

    n_pad = -(-n // 1024) * 1024
    sc = _make_sc_kernel(n, d_out, n_edges, sup_per_tile, n_pad)
    outp, denp = sc(h, asrc, adst, src, dst)

    denp3 = denp.reshape(NC, n_pad)[:, :n].reshape(NC, n, 1)
    bias2 = bias.reshape(1, d_out)
    out = pl.pallas_call(
        _final_body,
        grid=(n // bn,),
        in_specs=[
            pl.BlockSpec((NC, bn, d_out), lambda i: (0, i, 0)),
            pl.BlockSpec((NC, bn, 1), lambda i: (0, i, 0)),
            pl.BlockSpec((1, d_out), lambda i: (0, 0)),
        ],
        out_specs=pl.BlockSpec((bn, d_out), lambda i: (i, 0)),
        out_shape=jax.ShapeDtypeStruct((n, d_out), jnp.float32),
    )(outp, denp3, bias2)
    return out


# speedup vs baseline: 43.1766x; 1.0016x over previous
import functools

import jax
import jax.numpy as jnp
from jax import lax
from jax.experimental import pallas as pl
from jax.experimental.pallas import tpu as pltpu
from jax.experimental.pallas import tpu_sc as plsc

LANES = 16
CHUNK = 128
SUP = 16
NC = 2
NS = 16
NW = NC * NS


def _proj_body(x_ref, w_ref, att2_ref, h_ref, a2_ref):
    h = jnp.dot(x_ref[...], w_ref[...], preferred_element_type=jnp.float32)
    h_ref[...] = h
    a2_ref[...] = jnp.dot(h, att2_ref[...], preferred_element_type=jnp.float32)


def _final_body(o_ref, d_ref, b_ref, out_ref):
    o = o_ref[0] + o_ref[1]
    d = d_ref[0] + d_ref[1]
    v = o / (d + 1e-16) + b_ref[...]
    out_ref[...] = jnp.where(v > 0, v, jnp.exp(jnp.minimum(v, 0.0)) - 1.0)


def _make_sc_kernel(n_nodes, d_out, n_edges, sup_per_tile, n_pad):
    per_tile = sup_per_tile * SUP * CHUNK
    row_blk = (n_nodes // NS) // 8 * 8
    row_last = n_nodes - (NS - 1) * row_blk

    mesh = plsc.VectorSubcoreMesh(core_axis_name="c", subcore_axis_name="s")

    @functools.partial(
        pl.kernel,
        out_type=(
            jax.ShapeDtypeStruct((NC, n_nodes, d_out), jnp.float32),
            jax.ShapeDtypeStruct((NC * n_pad,), jnp.float32),
        ),
        mesh=mesh,
        compiler_params=pltpu.CompilerParams(needs_layout_passes=False),
        scratch_types=[
            pltpu.VMEM((2, SUP, CHUNK), jnp.int32),
            pltpu.VMEM((2, SUP, CHUNK), jnp.int32),
            pltpu.VMEM((2, CHUNK), jnp.float32),
            pltpu.VMEM((2, CHUNK), jnp.float32),
            pltpu.VMEM((2, CHUNK), jnp.float32),
            pltpu.VMEM((2, CHUNK, d_out), jnp.float32),
            pltpu.VMEM_SHARED((n_nodes + CHUNK, d_out), jnp.float32),
            pltpu.VMEM_SHARED((n_pad,), jnp.float32),
            pltpu.SemaphoreType.DMA,
            pltpu.SemaphoreType.DMA,
            pltpu.SemaphoreType.DMA,
            pltpu.SemaphoreType.DMA,
            pltpu.SemaphoreType.DMA,
            pltpu.SemaphoreType.DMA,
            pltpu.SemaphoreType.DMA,
            pltpu.SemaphoreType.DMA,
            pltpu.SemaphoreType.DMA,
            pltpu.SemaphoreType.DMA,
            pltpu.SemaphoreType.DMA,
        ],
    )
    def sc_kernel(h_hbm, asrc_hbm, adst_hbm, src_hbm, dst_hbm,
                  outp_hbm, denp_hbm,
                  sidx_v, didx_v, av_v, bv_v, ex_v, rows_v,
                  out_sh, den_sh, semidx,
                  semr0, semr1, sema0, sema1, semb0, semb1,
                  semd0, semd1, semo0, semo1):
        semr = (semr0, semr1)
        sema = (sema0, sema1)
        semb = (semb0, semb1)
        semd = (semd0, semd1)
        semo = (semo0, semo1)
        cid = lax.axis_index("c")
        sid = lax.axis_index("s")
        wid = cid * NS + sid

        zero_scope = jax.named_scope("sc_zero")
        zero_scope.__enter__()

        def _zrow(r, _):
            for k in range(d_out // LANES):
                rows_v[0, r, pl.ds(k * LANES, LANES)] = jnp.zeros(
                    (LANES,), jnp.float32)
            return 0
        lax.fori_loop(0, CHUNK, _zrow, 0)
        for g in range(CHUNK // LANES):
            ex_v[0, pl.ds(g * LANES, LANES)] = jnp.zeros(
                (LANES,), jnp.float32)

        def _zero_rows(start, count):
            off = 0
            while off < count:
                blk = min(CHUNK, count - off)
                pltpu.sync_copy(rows_v.at[0, pl.ds(0, blk)],
                                out_sh.at[pl.ds(start + off, blk)])
                off += blk

        @pl.when(sid < NS - 1)
        def _():
            _zero_rows(sid * row_blk, row_blk)

        @pl.when(sid == NS - 1)
        def _():
            _zero_rows((NS - 1) * row_blk, row_last)

        @pl.when(sid == 0)
        def _():
            for off2 in range(0, n_pad, CHUNK):
                pltpu.sync_copy(ex_v.at[0], den_sh.at[pl.ds(off2, CHUNK)])

        plsc.subcore_barrier()
        zero_scope.__exit__(None, None, None)

        base_edge = wid * per_tile
        n_blocks = sup_per_tile
        n_chunks = n_blocks * SUP

        def _idx_rows(t):
            q = (t // SUP) % 2
            r = t % SUP
            return sidx_v.at[q, r], didx_v.at[q, r]

        def _issue_gathers(t, p):
            s_row, d_row = _idx_rows(t)
            pltpu.async_copy(h_hbm.at[s_row], rows_v.at[p], semr[p])
            pltpu.async_copy(asrc_hbm.at[s_row], av_v.at[p], sema[p])
            pltpu.async_copy(adst_hbm.at[d_row], bv_v.at[p], semb[p])

        def _wait_scat(p):
            s_row, d_row = _idx_rows(0)
            pltpu.make_async_copy(ex_v.at[p], den_sh.at[d_row],
                                  semd[p]).wait()
            pltpu.make_async_copy(rows_v.at[p], out_sh.at[d_row],
                                  semo[p]).wait()

        def _process(t, p):
            s_row, d_row = _idx_rows(t)
            pltpu.make_async_copy(asrc_hbm.at[s_row], av_v.at[p],
                                  sema[p]).wait()
            pltpu.make_async_copy(adst_hbm.at[d_row], bv_v.at[p],
                                  semb[p]).wait()

            def g_body(g, _):
                e = (av_v[p, pl.ds(g * LANES, LANES)]
                     + bv_v[p, pl.ds(g * LANES, LANES)])
                e = jnp.where(e >= 0.0, e, 0.2 * e)
                ex = jnp.exp(e)
                gid = (base_edge + t * CHUNK + g * LANES
                       + lax.iota(jnp.int32, 16))
                ex = jnp.where(gid < n_edges, ex, 0.0)
                ex_v[p, pl.ds(g * LANES, LANES)] = ex
                return 0
            lax.fori_loop(0, CHUNK // LANES, g_body, 0)

            pltpu.async_copy(ex_v.at[p], den_sh.at[d_row], semd[p],
                             add=True)

            pltpu.make_async_copy(h_hbm.at[s_row], rows_v.at[p],
                                  semr[p]).wait()

            def m_body(g, _):
                for j in range(LANES):
                    rr = g * LANES + j
                    sp = plsc.load_gather(
                        ex_v.at[p], [jnp.full((LANES,), rr, jnp.int32)])
                    for k in range(d_out // LANES):
                        rows_v[p, rr, pl.ds(k * LANES, LANES)] = (
                            rows_v[p, rr, pl.ds(k * LANES, LANES)] * sp)
                return 0
            lax.fori_loop(0, CHUNK // LANES, m_body, 0)

            pltpu.async_copy(rows_v.at[p], out_sh.at[d_row], semo[p],
                             add=True)

        def _wait_idx_block(b):
            qb = b % 2
            pltpu.make_async_copy(src_hbm.at[wid, b], sidx_v.at[qb],
                                  semidx).wait()
            pltpu.make_async_copy(dst_hbm.at[wid, b], didx_v.at[qb],
                                  semidx).wait()

        with jax.named_scope("sc_prologue"):
            pltpu.sync_copy(src_hbm.at[wid, 0], sidx_v.at[0])
            pltpu.sync_copy(dst_hbm.at[wid, 0], didx_v.at[0])
            _issue_gathers(0, 0)

        def pipe_body(u, _):
            ta = 2 * u
            tb = ta + 1
            @pl.when(ta % SUP == 0)
            def _():
                b = ta // SUP

                @pl.when(b + 1 < n_blocks)
                def _():
                    qn = (b + 1) % 2
                    pltpu.async_copy(src_hbm.at[wid, b + 1], sidx_v.at[qn],
                                     semidx)
                    pltpu.async_copy(dst_hbm.at[wid, b + 1], didx_v.at[qn],
                                     semidx)

            @pl.when(u > 0)
            def _():
                _wait_scat(1)
            _issue_gathers(tb, 1)

            _process(ta, 0)
            _process(tb, 1)

            tn = ta + 2

            @pl.when(tn < n_chunks)
            def _():
                @pl.when(tn % SUP == 0)
                def _():
                    _wait_idx_block(tn // SUP)
                _wait_scat(0)
                _issue_gathers(tn, 0)
            return 0

        with jax.named_scope("sc_mainloop"):
            lax.fori_loop(0, n_chunks // 2, pipe_body, 0)

        with jax.named_scope("sc_epilogue"):
            _wait_scat(0)
            _wait_scat(1)

            plsc.subcore_barrier()

        with jax.named_scope("sc_drain"):
            @pl.when(sid < NS - 1)
            def _():
                pltpu.sync_copy(
                    out_sh.at[pl.ds(sid * row_blk, row_blk)],
                    outp_hbm.at[cid, pl.ds(sid * row_blk, row_blk)])

            @pl.when(sid == NS - 1)
            def _():
                pltpu.sync_copy(
                    out_sh.at[pl.ds((NS - 1) * row_blk, row_last)],
                    outp_hbm.at[cid, pl.ds((NS - 1) * row_blk, row_last)])

            @pl.when(sid == 0)
            def _():
                pltpu.sync_copy(den_sh,
                                denp_hbm.at[pl.ds(cid * n_pad, n_pad)])

    return sc_kernel


def kernel(input, edge_index, W, att_src, att_dst, bias):
    n, d_in = input.shape
    d_out = W.shape[1]
    n_edges = edge_index.shape[1]

    bn = 1000
    att2 = jnp.stack([att_src, att_dst], axis=1)
    h, a2 = pl.pallas_call(
        _proj_body,
        grid=(n // bn,),
        in_specs=[
            pl.BlockSpec((bn, d_in), lambda i: (i, 0)),
            pl.BlockSpec((d_in, d_out), lambda i: (0, 0)),
            pl.BlockSpec((d_out, 2), lambda i: (0, 0)),
        ],
        out_specs=[
            pl.BlockSpec((bn, d_out), lambda i: (i, 0)),
            pl.BlockSpec((bn, 2), lambda i: (i, 0)),
        ],
        out_shape=[
            jax.ShapeDtypeStruct((n, d_out), jnp.float32),
            jax.ShapeDtypeStruct((n, 2), jnp.float32),
        ],
    )(input, W, att2)
    asrc = a2[:, 0]
    adst = a2[:, 1]

    sup_per_tile = -(-n_edges // (NW * SUP * CHUNK))
    e_pad = NW * sup_per_tile * SUP * CHUNK
    src = edge_index[0].astype(jnp.int32)
    dst = edge_index[1].astype(jnp.int32)
    npad_e = e_pad - n_edges
    pad_iota = lax.iota(jnp.int32, npad_e)
    src = jnp.concatenate([src, pad_iota % n]).reshape(
        NW, sup_per_tile, SUP, CHUNK)
    dst = jnp.concatenate([dst, n + (pad_iota % CHUNK)]).reshape(
        NW, sup_per_tile, SUP, CHUNK)

    n_pad = -(-n // 1024) * 1024
    sc = _make_sc_kernel(n, d_out, n_edges, sup_per_tile, n_pad)
    outp, denp = sc(h, asrc, adst, src, dst)

    denp3 = denp.reshape(NC, n_pad)[:, :n].reshape(NC, n, 1)
    bias2 = bias.reshape(1, d_out)
    out = pl.pallas_call(
        _final_body,
        grid=(n // bn,),
        in_specs=[
            pl.BlockSpec((NC, bn, d_out), lambda i: (0, i, 0)),
            pl.BlockSpec((NC, bn, 1), lambda i: (0, i, 0)),
            pl.BlockSpec((1, d_out), lambda i: (0, 0)),
        ],
        out_specs=pl.BlockSpec((bn, d_out), lambda i: (i, 0)),
        out_shape=jax.ShapeDtypeStruct((n, d_out), jnp.float32),
    )(outp, denp3, bias2)
    return out


# speedup vs baseline: 43.2333x; 1.0013x over previous
import functools

import jax
import jax.numpy as jnp
from jax import lax
from jax.experimental import pallas as pl
from jax.experimental.pallas import tpu as pltpu
from jax.experimental.pallas import tpu_sc as plsc

LANES = 16
CHUNK = 64
SUP = 16
NC = 2
NS = 16
NW = NC * NS


def _proj_body(x_ref, w_ref, att2_ref, h_ref, a2_ref):
    h = jnp.dot(x_ref[...], w_ref[...], preferred_element_type=jnp.float32)
    h_ref[...] = h
    a2_ref[...] = jnp.dot(h, att2_ref[...], preferred_element_type=jnp.float32)


def _final_body(o_ref, d_ref, b_ref, out_ref):
    o = o_ref[0] + o_ref[1]
    d = d_ref[0] + d_ref[1]
    v = o / (d + 1e-16) + b_ref[...]
    out_ref[...] = jnp.where(v > 0, v, jnp.exp(jnp.minimum(v, 0.0)) - 1.0)


def _make_sc_kernel(n_nodes, d_out, n_edges, sup_per_tile, n_pad):
    per_tile = sup_per_tile * SUP * CHUNK
    row_blk = (n_nodes // NS) // 8 * 8
    row_last = n_nodes - (NS - 1) * row_blk

    mesh = plsc.VectorSubcoreMesh(core_axis_name="c", subcore_axis_name="s")

    @functools.partial(
        pl.kernel,
        out_type=(
            jax.ShapeDtypeStruct((NC, n_nodes, d_out), jnp.float32),
            jax.ShapeDtypeStruct((NC * n_pad,), jnp.float32),
        ),
        mesh=mesh,
        compiler_params=pltpu.CompilerParams(needs_layout_passes=False),
        scratch_types=[
            pltpu.VMEM((2, SUP, CHUNK), jnp.int32),
            pltpu.VMEM((2, SUP, CHUNK), jnp.int32),
            pltpu.VMEM((2, CHUNK), jnp.float32),
            pltpu.VMEM((2, CHUNK), jnp.float32),
            pltpu.VMEM((2, CHUNK), jnp.float32),
            pltpu.VMEM((2, CHUNK, d_out), jnp.float32),
            pltpu.VMEM((CHUNK, d_out), jnp.float32),
            pltpu.VMEM_SHARED((n_nodes + CHUNK, d_out), jnp.float32),
            pltpu.VMEM_SHARED((n_pad,), jnp.float32),
            pltpu.SemaphoreType.DMA,
            pltpu.SemaphoreType.DMA,
            pltpu.SemaphoreType.DMA,
            pltpu.SemaphoreType.DMA,
            pltpu.SemaphoreType.DMA,
            pltpu.SemaphoreType.DMA,
            pltpu.SemaphoreType.DMA,
            pltpu.SemaphoreType.DMA,
            pltpu.SemaphoreType.DMA,
            pltpu.SemaphoreType.DMA,
        ],
    )
    def sc_kernel(h_hbm, asrc_hbm, adst_hbm, src_hbm, dst_hbm,
                  outp_hbm, denp_hbm,
                  sidx_v, didx_v, av_v, bv_v, ex_v, rows_v, srows_v,
                  out_sh, den_sh, semidx,
                  semr0, semr1, sema0, sema1, semb0, semb1,
                  semd0, semd1, semo):
        semr = (semr0, semr1)
        sema = (sema0, sema1)
        semb = (semb0, semb1)
        semd = (semd0, semd1)
        cid = lax.axis_index("c")
        sid = lax.axis_index("s")
        wid = cid * NS + sid

        zero_scope = jax.named_scope("sc_zero")
        zero_scope.__enter__()

        def _zrow(r, _):
            for k in range(d_out // LANES):
                srows_v[r, pl.ds(k * LANES, LANES)] = jnp.zeros(
                    (LANES,), jnp.float32)
            return 0
        lax.fori_loop(0, CHUNK, _zrow, 0)
        for g in range(CHUNK // LANES):
            ex_v[0, pl.ds(g * LANES, LANES)] = jnp.zeros(
                (LANES,), jnp.float32)

        def _zero_rows(start, count):
            off = 0
            while off < count:
                blk = min(CHUNK, count - off)
                pltpu.sync_copy(srows_v.at[pl.ds(0, blk)],
                                out_sh.at[pl.ds(start + off, blk)])
                off += blk

        @pl.when(sid < NS - 1)
        def _():
            _zero_rows(sid * row_blk, row_blk)

        @pl.when(sid == NS - 1)
        def _():
            _zero_rows((NS - 1) * row_blk, row_last)

        @pl.when(sid == 0)
        def _():
            for off2 in range(0, n_pad, CHUNK):
                pltpu.sync_copy(ex_v.at[0], den_sh.at[pl.ds(off2, CHUNK)])

        plsc.subcore_barrier()
        zero_scope.__exit__(None, None, None)

        base_edge = wid * per_tile
        n_blocks = sup_per_tile
        n_chunks = n_blocks * SUP

        def _idx_rows(t):
            q = (t // SUP) % 2
            r = t % SUP
            return sidx_v.at[q, r], didx_v.at[q, r]

        def _issue_gathers(t, p):
            s_row, d_row = _idx_rows(t)
            pltpu.async_copy(h_hbm.at[s_row], rows_v.at[p], semr[p])
            pltpu.async_copy(asrc_hbm.at[s_row], av_v.at[p], sema[p])
            pltpu.async_copy(adst_hbm.at[d_row], bv_v.at[p], semb[p])

        def _wait_den_scat(p):
            s_row, d_row = _idx_rows(0)
            pltpu.make_async_copy(ex_v.at[p], den_sh.at[d_row],
                                  semd[p]).wait()

        def _wait_out_scat():
            s_row, d_row = _idx_rows(0)
            pltpu.make_async_copy(srows_v, out_sh.at[d_row], semo).wait()

        def _process(t, p):
            s_row, d_row = _idx_rows(t)
            @pl.when(t > 1)
            def _():
                _wait_den_scat(p)
            pltpu.make_async_copy(asrc_hbm.at[s_row], av_v.at[p],
                                  sema[p]).wait()
            pltpu.make_async_copy(adst_hbm.at[d_row], bv_v.at[p],
                                  semb[p]).wait()

            def g_body(g, _):
                e = (av_v[p, pl.ds(g * LANES, LANES)]
                     + bv_v[p, pl.ds(g * LANES, LANES)])
                e = jnp.where(e >= 0.0, e, 0.2 * e)
                ex = jnp.exp(e)
                gid = (base_edge + t * CHUNK + g * LANES
                       + lax.iota(jnp.int32, 16))
                ex = jnp.where(gid < n_edges, ex, 0.0)
                ex_v[p, pl.ds(g * LANES, LANES)] = ex
                return 0
            lax.fori_loop(0, CHUNK // LANES, g_body, 0)

            pltpu.async_copy(ex_v.at[p], den_sh.at[d_row], semd[p],
                             add=True)

            pltpu.make_async_copy(h_hbm.at[s_row], rows_v.at[p],
                                  semr[p]).wait()

            @pl.when(t > 0)
            def _():
                _wait_out_scat()

            def m_body(g, _):
                for j in range(LANES):
                    rr = g * LANES + j
                    sp = plsc.load_gather(
                        ex_v.at[p], [jnp.full((LANES,), rr, jnp.int32)])
                    for k in range(d_out // LANES):
                        srows_v[rr, pl.ds(k * LANES, LANES)] = (
                            rows_v[p, rr, pl.ds(k * LANES, LANES)] * sp)
                return 0
            lax.fori_loop(0, CHUNK // LANES, m_body, 0)

            pltpu.async_copy(srows_v, out_sh.at[d_row], semo, add=True)

            tn = t + 2

            @pl.when(tn < n_chunks)
            def _():
                @pl.when(tn % SUP == 0)
                def _():
                    _wait_idx_block(tn // SUP)
                _issue_gathers(tn, p)

        def _wait_idx_block(b):
            qb = b % 2
            pltpu.make_async_copy(src_hbm.at[wid, b], sidx_v.at[qb],
                                  semidx).wait()
            pltpu.make_async_copy(dst_hbm.at[wid, b], didx_v.at[qb],
                                  semidx).wait()

        with jax.named_scope("sc_prologue"):
            pltpu.sync_copy(src_hbm.at[wid, 0], sidx_v.at[0])
            pltpu.sync_copy(dst_hbm.at[wid, 0], didx_v.at[0])
            _issue_gathers(0, 0)
            _issue_gathers(1, 1)

        def pipe_body(u, _):
            ta = 2 * u
            @pl.when(ta % SUP == 0)
            def _():
                b = ta // SUP

                @pl.when(b + 1 < n_blocks)
                def _():
                    qn = (b + 1) % 2
                    pltpu.async_copy(src_hbm.at[wid, b + 1], sidx_v.at[qn],
                                     semidx)
                    pltpu.async_copy(dst_hbm.at[wid, b + 1], didx_v.at[qn],
                                     semidx)

            _process(ta, 0)
            _process(ta + 1, 1)
            return 0

        with jax.named_scope("sc_mainloop"):
            lax.fori_loop(0, n_chunks // 2, pipe_body, 0)

        with jax.named_scope("sc_epilogue"):
            _wait_den_scat(0)
            _wait_den_scat(1)
            _wait_out_scat()

            plsc.subcore_barrier()

        with jax.named_scope("sc_drain"):
            @pl.when(sid < NS - 1)
            def _():
                pltpu.sync_copy(
                    out_sh.at[pl.ds(sid * row_blk, row_blk)],
                    outp_hbm.at[cid, pl.ds(sid * row_blk, row_blk)])

            @pl.when(sid == NS - 1)
            def _():
                pltpu.sync_copy(
                    out_sh.at[pl.ds((NS - 1) * row_blk, row_last)],
                    outp_hbm.at[cid, pl.ds((NS - 1) * row_blk, row_last)])

            @pl.when(sid == 0)
            def _():
                pltpu.sync_copy(den_sh,
                                denp_hbm.at[pl.ds(cid * n_pad, n_pad)])

    return sc_kernel


def kernel(input, edge_index, W, att_src, att_dst, bias):
    n, d_in = input.shape
    d_out = W.shape[1]
    n_edges = edge_index.shape[1]

    bn = 1000
    att2 = jnp.stack([att_src, att_dst], axis=1)
    h, a2 = pl.pallas_call(
        _proj_body,
        grid=(n // bn,),
        in_specs=[
            pl.BlockSpec((bn, d_in), lambda i: (i, 0)),
            pl.BlockSpec((d_in, d_out), lambda i: (0, 0)),
            pl.BlockSpec((d_out, 2), lambda i: (0, 0)),
        ],
        out_specs=[
            pl.BlockSpec((bn, d_out), lambda i: (i, 0)),
            pl.BlockSpec((bn, 2), lambda i: (i, 0)),
        ],
        out_shape=[
            jax.ShapeDtypeStruct((n, d_out), jnp.float32),
            jax.ShapeDtypeStruct((n, 2), jnp.float32),
        ],
    )(input, W, att2)
    asrc = a2[:, 0]
    adst = a2[:, 1]

    sup_per_tile = -(-n_edges // (NW * SUP * CHUNK))
    e_pad = NW * sup_per_tile * SUP * CHUNK
    src = edge_index[0].astype(jnp.int32)
    dst = edge_index[1].astype(jnp.int32)
    npad_e = e_pad - n_edges
    pad_iota = lax.iota(jnp.int32, npad_e)
    src = jnp.concatenate([src, pad_iota % n]).reshape(
        NW, sup_per_tile, SUP, CHUNK)
    dst = jnp.concatenate([dst, n + (pad_iota % CHUNK)]).reshape(
        NW, sup_per_tile, SUP, CHUNK)

    n_pad = -(-n // 1024) * 1024
    sc = _make_sc_kernel(n, d_out, n_edges, sup_per_tile, n_pad)
    outp, denp = sc(h, asrc, adst, src, dst)

    denp3 = denp.reshape(NC, n_pad)[:, :n].reshape(NC, n, 1)
    bias2 = bias.reshape(1, d_out)
    out = pl.pallas_call(
        _final_body,
        grid=(n // bn,),
        in_specs=[
            pl.BlockSpec((NC, bn, d_out), lambda i: (0, i, 0)),
            pl.BlockSpec((NC, bn, 1), lambda i: (0, i, 0)),
            pl.BlockSpec((1, d_out), lambda i: (0, 0)),
        ],
        out_specs=pl.BlockSpec((bn, d_out), lambda i: (i, 0)),
        out_shape=jax.ShapeDtypeStruct((n, d_out), jnp.float32),
    )(outp, denp3, bias2)
    return out


# speedup vs baseline: 44.1781x; 1.0219x over previous
import functools

import jax
import jax.numpy as jnp
from jax import lax
from jax.experimental import pallas as pl
from jax.experimental.pallas import tpu as pltpu
from jax.experimental.pallas import tpu_sc as plsc

LANES = 16
CHUNK = 64
SUP = 16
NC = 2
NS = 16
NW = NC * NS


def _proj_body(x_ref, w_ref, att2_ref, h_ref, a2_ref):
    h = jnp.dot(x_ref[...], w_ref[...], preferred_element_type=jnp.float32)
    h_ref[...] = h
    a2_ref[...] = jnp.dot(h, att2_ref[...], preferred_element_type=jnp.float32)


def _final_body(o_ref, d_ref, b_ref, out_ref):
    o = o_ref[0] + o_ref[1]
    d = d_ref[0] + d_ref[1]
    v = o / (d + 1e-16) + b_ref[...]
    out_ref[...] = jnp.where(v > 0, v, jnp.exp(jnp.minimum(v, 0.0)) - 1.0)


def _make_sc_kernel(n_nodes, d_out, n_edges, sup_per_tile, n_pad):
    per_tile = sup_per_tile * SUP * CHUNK
    row_blk = (n_nodes // NS) // 8 * 8
    row_last = n_nodes - (NS - 1) * row_blk

    mesh = plsc.VectorSubcoreMesh(core_axis_name="c", subcore_axis_name="s")

    @functools.partial(
        pl.kernel,
        out_type=(
            jax.ShapeDtypeStruct((NC, n_nodes, d_out), jnp.float32),
            jax.ShapeDtypeStruct((NC * n_pad,), jnp.float32),
        ),
        mesh=mesh,
        compiler_params=pltpu.CompilerParams(needs_layout_passes=False),
        scratch_types=[
            pltpu.VMEM((2, SUP, CHUNK), jnp.int32),
            pltpu.VMEM((2, SUP, CHUNK), jnp.int32),
            pltpu.VMEM((2, CHUNK), jnp.float32),
            pltpu.VMEM((2, CHUNK), jnp.float32),
            pltpu.VMEM((2, CHUNK), jnp.float32),
            pltpu.VMEM((2, CHUNK, d_out), jnp.float32),
            pltpu.VMEM((CHUNK, d_out), jnp.float32),
            pltpu.VMEM_SHARED((n_nodes + CHUNK, d_out), jnp.float32),
            pltpu.VMEM_SHARED((n_pad,), jnp.float32),
            pltpu.SemaphoreType.DMA,
            pltpu.SemaphoreType.DMA,
            pltpu.SemaphoreType.DMA,
            pltpu.SemaphoreType.DMA,
            pltpu.SemaphoreType.DMA,
            pltpu.SemaphoreType.DMA,
            pltpu.SemaphoreType.DMA,
            pltpu.SemaphoreType.DMA,
            pltpu.SemaphoreType.DMA,
            pltpu.SemaphoreType.DMA,
        ],
    )
    def sc_kernel(h_hbm, asrc_hbm, adst_hbm, ei_hbm,
                  outp_hbm, denp_hbm,
                  sidx_v, didx_v, av_v, bv_v, ex_v, rows_v, srows_v,
                  out_sh, den_sh, semidx,
                  semr0, semr1, sema0, sema1, semb0, semb1,
                  semd0, semd1, semo):
        semr = (semr0, semr1)
        sema = (sema0, sema1)
        semb = (semb0, semb1)
        semd = (semd0, semd1)
        cid = lax.axis_index("c")
        sid = lax.axis_index("s")
        wid = cid * NS + sid

        zero_scope = jax.named_scope("sc_zero")
        zero_scope.__enter__()

        def _zrow(r, _):
            for k in range(d_out // LANES):
                srows_v[r, pl.ds(k * LANES, LANES)] = jnp.zeros(
                    (LANES,), jnp.float32)
            return 0
        lax.fori_loop(0, CHUNK, _zrow, 0)
        for g in range(CHUNK // LANES):
            ex_v[0, pl.ds(g * LANES, LANES)] = jnp.zeros(
                (LANES,), jnp.float32)

        def _zero_rows(start, count):
            off = 0
            while off < count:
                blk = min(CHUNK, count - off)
                pltpu.sync_copy(srows_v.at[pl.ds(0, blk)],
                                out_sh.at[pl.ds(start + off, blk)])
                off += blk

        @pl.when(sid < NS - 1)
        def _():
            _zero_rows(sid * row_blk, row_blk)

        @pl.when(sid == NS - 1)
        def _():
            _zero_rows((NS - 1) * row_blk, row_last)

        @pl.when(sid == 0)
        def _():
            for off2 in range(0, n_pad, CHUNK):
                pltpu.sync_copy(ex_v.at[0], den_sh.at[pl.ds(off2, CHUNK)])

        plsc.subcore_barrier()
        zero_scope.__exit__(None, None, None)

        base_edge = wid * per_tile
        n_blocks = sup_per_tile
        n_chunks = n_blocks * SUP

        def _idx_rows(t):
            q = (t // SUP) % 2
            r = t % SUP
            return sidx_v.at[q, r], didx_v.at[q, r]

        def _issue_gathers(t, p):
            s_row, d_row = _idx_rows(t)
            pltpu.async_copy(h_hbm.at[s_row], rows_v.at[p], semr[p])
            pltpu.async_copy(asrc_hbm.at[s_row], av_v.at[p], sema[p])
            pltpu.async_copy(adst_hbm.at[d_row], bv_v.at[p], semb[p])

        def _wait_den_scat(p):
            s_row, d_row = _idx_rows(0)
            pltpu.make_async_copy(ex_v.at[p], den_sh.at[d_row],
                                  semd[p]).wait()

        def _wait_out_scat():
            s_row, d_row = _idx_rows(0)
            pltpu.make_async_copy(srows_v, out_sh.at[d_row], semo).wait()

        def _process(t, p):
            s_row, d_row = _idx_rows(t)
            @pl.when(t > 1)
            def _():
                _wait_den_scat(p)
            pltpu.make_async_copy(asrc_hbm.at[s_row], av_v.at[p],
                                  sema[p]).wait()
            pltpu.make_async_copy(adst_hbm.at[d_row], bv_v.at[p],
                                  semb[p]).wait()

            def g_body(g, _):
                e = (av_v[p, pl.ds(g * LANES, LANES)]
                     + bv_v[p, pl.ds(g * LANES, LANES)])
                e = jnp.where(e >= 0.0, e, 0.2 * e)
                ex = jnp.exp(e)
                gid = (base_edge + t * CHUNK + g * LANES
                       + lax.iota(jnp.int32, 16))
                ex = jnp.where(gid < n_edges, ex, 0.0)
                ex_v[p, pl.ds(g * LANES, LANES)] = ex
                return 0
            lax.fori_loop(0, CHUNK // LANES, g_body, 0)

            pltpu.async_copy(ex_v.at[p], den_sh.at[d_row], semd[p],
                             add=True)

            pltpu.make_async_copy(h_hbm.at[s_row], rows_v.at[p],
                                  semr[p]).wait()

            @pl.when(t > 0)
            def _():
                _wait_out_scat()

            def m_body(g, _):
                for j in range(LANES):
                    rr = g * LANES + j
                    sp = plsc.load_gather(
                        ex_v.at[p], [jnp.full((LANES,), rr, jnp.int32)])
                    for k in range(d_out // LANES):
                        srows_v[rr, pl.ds(k * LANES, LANES)] = (
                            rows_v[p, rr, pl.ds(k * LANES, LANES)] * sp)
                return 0
            lax.fori_loop(0, CHUNK // LANES, m_body, 0)

            pltpu.async_copy(srows_v, out_sh.at[d_row], semo, add=True)

            tn = t + 2

            @pl.when(tn < n_chunks)
            def _():
                @pl.when(tn % SUP == 0)
                def _():
                    _wait_idx_block(tn // SUP)
                _issue_gathers(tn, p)

        def _wait_idx_block(b):
            qb = b % 2
            pltpu.make_async_copy(ei_hbm.at[0, wid, b], sidx_v.at[qb],
                                  semidx).wait()
            pltpu.make_async_copy(ei_hbm.at[1, wid, b], didx_v.at[qb],
                                  semidx).wait()

        with jax.named_scope("sc_prologue"):
            pltpu.sync_copy(ei_hbm.at[0, wid, 0], sidx_v.at[0])
            pltpu.sync_copy(ei_hbm.at[1, wid, 0], didx_v.at[0])
            _issue_gathers(0, 0)
            _issue_gathers(1, 1)

        def pipe_body(u, _):
            ta = 2 * u
            @pl.when(ta % SUP == 0)
            def _():
                b = ta // SUP

                @pl.when(b + 1 < n_blocks)
                def _():
                    qn = (b + 1) % 2
                    pltpu.async_copy(ei_hbm.at[0, wid, b + 1], sidx_v.at[qn],
                                     semidx)
                    pltpu.async_copy(ei_hbm.at[1, wid, b + 1], didx_v.at[qn],
                                     semidx)

            _process(ta, 0)
            _process(ta + 1, 1)
            return 0

        with jax.named_scope("sc_mainloop"):
            lax.fori_loop(0, n_chunks // 2, pipe_body, 0)

        with jax.named_scope("sc_epilogue"):
            _wait_den_scat(0)
            _wait_den_scat(1)
            _wait_out_scat()

            plsc.subcore_barrier()

        with jax.named_scope("sc_drain"):
            @pl.when(sid < NS - 1)
            def _():
                pltpu.sync_copy(
                    out_sh.at[pl.ds(sid * row_blk, row_blk)],
                    outp_hbm.at[cid, pl.ds(sid * row_blk, row_blk)])

            @pl.when(sid == NS - 1)
            def _():
                pltpu.sync_copy(
                    out_sh.at[pl.ds((NS - 1) * row_blk, row_last)],
                    outp_hbm.at[cid, pl.ds((NS - 1) * row_blk, row_last)])

            @pl.when(sid == 0)
            def _():
                pltpu.sync_copy(den_sh,
                                denp_hbm.at[pl.ds(cid * n_pad, n_pad)])

    return sc_kernel


def kernel(input, edge_index, W, att_src, att_dst, bias):
    n, d_in = input.shape
    d_out = W.shape[1]
    n_edges = edge_index.shape[1]

    bn = 1000
    att2 = jnp.stack([att_src, att_dst], axis=1)
    h, a2 = pl.pallas_call(
        _proj_body,
        grid=(n // bn,),
        in_specs=[
            pl.BlockSpec((bn, d_in), lambda i: (i, 0)),
            pl.BlockSpec((d_in, d_out), lambda i: (0, 0)),
            pl.BlockSpec((d_out, 2), lambda i: (0, 0)),
        ],
        out_specs=[
            pl.BlockSpec((bn, d_out), lambda i: (i, 0)),
            pl.BlockSpec((bn, 2), lambda i: (i, 0)),
        ],
        out_shape=[
            jax.ShapeDtypeStruct((n, d_out), jnp.float32),
            jax.ShapeDtypeStruct((n, 2), jnp.float32),
        ],
    )(input, W, att2)
    asrc = a2[:, 0]
    adst = a2[:, 1]

    sup_per_tile = -(-n_edges // (NW * SUP * CHUNK))
    e_pad = NW * sup_per_tile * SUP * CHUNK
    npad_e = e_pad - n_edges
    pad_iota = lax.iota(jnp.int32, npad_e)
    pad_block = jnp.stack([pad_iota % n, n + (pad_iota % CHUNK)])
    ei = jnp.concatenate([edge_index.astype(jnp.int32), pad_block], axis=1)
    ei = ei.reshape(2, NW, sup_per_tile, SUP, CHUNK)

    n_pad = -(-n // 1024) * 1024
    sc = _make_sc_kernel(n, d_out, n_edges, sup_per_tile, n_pad)
    outp, denp = sc(h, asrc, adst, ei)

    denp3 = denp.reshape(NC, n_pad)[:, :n].reshape(NC, n, 1)
    bias2 = bias.reshape(1, d_out)
    out = pl.pallas_call(
        _final_body,
        grid=(n // bn,),
        in_specs=[
            pl.BlockSpec((NC, bn, d_out), lambda i: (0, i, 0)),
            pl.BlockSpec((NC, bn, 1), lambda i: (0, i, 0)),
            pl.BlockSpec((1, d_out), lambda i: (0, 0)),
        ],
        out_specs=pl.BlockSpec((bn, d_out), lambda i: (i, 0)),
        out_shape=jax.ShapeDtypeStruct((n, d_out), jnp.float32),
    )(outp, denp3, bias2)
    return out


# speedup vs baseline: 46.9093x; 1.0618x over previous
import functools

import jax
import jax.numpy as jnp
from jax import lax
from jax.experimental import pallas as pl
from jax.experimental.pallas import tpu as pltpu
from jax.experimental.pallas import tpu_sc as plsc

LANES = 16
CHUNK = 64
SUP = 16
NC = 2
NS = 16
NW = NC * NS


def _proj_body(x_ref, w_ref, att2_ref, h_ref, a2_ref):
    h = jnp.dot(x_ref[...], w_ref[...], preferred_element_type=jnp.float32)
    h_ref[...] = h
    a2_ref[...] = jnp.dot(h, att2_ref[...], preferred_element_type=jnp.float32)


def _final_body(o_ref, d_ref, b_ref, out_ref):
    o = o_ref[0] + o_ref[1]
    d = d_ref[0] + d_ref[1]
    v = o / (d + 1e-16) + b_ref[...]
    out_ref[...] = jnp.where(v > 0, v, jnp.exp(jnp.minimum(v, 0.0)) - 1.0)


def _make_sc_kernel(n_nodes, d_out, n_edges, sup_per_tile, n_pad):
    per_tile = sup_per_tile * SUP * CHUNK
    row_blk = (n_nodes // NS) // 8 * 8
    row_last = n_nodes - (NS - 1) * row_blk

    mesh = plsc.VectorSubcoreMesh(core_axis_name="c", subcore_axis_name="s")

    @functools.partial(
        pl.kernel,
        out_type=(
            jax.ShapeDtypeStruct((NC, n_nodes, d_out), jnp.float32),
            jax.ShapeDtypeStruct((NC * n_pad,), jnp.float32),
        ),
        mesh=mesh,
        compiler_params=pltpu.CompilerParams(needs_layout_passes=False),
        scratch_types=[
            pltpu.VMEM((2, SUP, CHUNK), jnp.int32),
            pltpu.VMEM((2, SUP, CHUNK), jnp.int32),
            pltpu.VMEM((2, CHUNK), jnp.float32),
            pltpu.VMEM((2, CHUNK), jnp.float32),
            pltpu.VMEM((2, CHUNK), jnp.float32),
            pltpu.VMEM((2, CHUNK, d_out), jnp.float32),
            pltpu.VMEM((CHUNK, d_out), jnp.float32),
            pltpu.VMEM_SHARED((n_nodes + CHUNK, d_out), jnp.float32),
            pltpu.VMEM_SHARED((n_pad,), jnp.float32),
            pltpu.SemaphoreType.DMA,
            pltpu.SemaphoreType.DMA,
            pltpu.SemaphoreType.DMA,
            pltpu.SemaphoreType.DMA,
            pltpu.SemaphoreType.DMA,
            pltpu.SemaphoreType.DMA,
            pltpu.SemaphoreType.DMA,
            pltpu.SemaphoreType.DMA,
            pltpu.SemaphoreType.DMA,
            pltpu.SemaphoreType.DMA,
        ],
    )
    def sc_kernel(h_hbm, asrc_hbm, adst_hbm, ei_hbm,
                  outp_hbm, denp_hbm,
                  sidx_v, didx_v, av_v, bv_v, ex_v, rows_v, srows_v,
                  out_sh, den_sh, semidx,
                  semr0, semr1, sema0, sema1, semb0, semb1,
                  semd0, semd1, semo):
        semr = (semr0, semr1)
        sema = (sema0, sema1)
        semb = (semb0, semb1)
        semd = (semd0, semd1)
        cid = lax.axis_index("c")
        sid = lax.axis_index("s")
        wid = cid * NS + sid

        zero_scope = jax.named_scope("sc_zero")
        zero_scope.__enter__()

        def _zrow(r, _):
            for k in range(d_out // LANES):
                srows_v[r, pl.ds(k * LANES, LANES)] = jnp.zeros(
                    (LANES,), jnp.float32)
            return 0
        lax.fori_loop(0, CHUNK, _zrow, 0)
        for g in range(CHUNK // LANES):
            ex_v[0, pl.ds(g * LANES, LANES)] = jnp.zeros(
                (LANES,), jnp.float32)

        def _zero_rows(start, count):
            off = 0
            while off < count:
                blk = min(CHUNK, count - off)
                pltpu.sync_copy(srows_v.at[pl.ds(0, blk)],
                                out_sh.at[pl.ds(start + off, blk)])
                off += blk

        @pl.when(sid < NS - 1)
        def _():
            _zero_rows(sid * row_blk, row_blk)

        @pl.when(sid == NS - 1)
        def _():
            _zero_rows((NS - 1) * row_blk, row_last)

        den_per_tile = n_pad // NS
        den_base = sid * den_per_tile
        for off2 in range(0, den_per_tile, CHUNK):
            pltpu.sync_copy(ex_v.at[0],
                            den_sh.at[pl.ds(den_base + off2, CHUNK)])

        plsc.subcore_barrier()
        zero_scope.__exit__(None, None, None)

        base_edge = wid * per_tile
        n_blocks = sup_per_tile
        n_chunks = n_blocks * SUP

        def _idx_rows(t):
            q = (t // SUP) % 2
            r = t % SUP
            return sidx_v.at[q, r], didx_v.at[q, r]

        def _issue_gathers(t, p):
            s_row, d_row = _idx_rows(t)
            pltpu.async_copy(h_hbm.at[s_row], rows_v.at[p], semr[p])
            pltpu.async_copy(asrc_hbm.at[s_row], av_v.at[p], sema[p])
            pltpu.async_copy(adst_hbm.at[d_row], bv_v.at[p], semb[p])

        def _wait_den_scat(p):
            s_row, d_row = _idx_rows(0)
            pltpu.make_async_copy(ex_v.at[p], den_sh.at[d_row],
                                  semd[p]).wait()

        def _wait_out_scat():
            s_row, d_row = _idx_rows(0)
            pltpu.make_async_copy(srows_v, out_sh.at[d_row], semo).wait()

        def _process(t, p):
            s_row, d_row = _idx_rows(t)
            @pl.when(t > 1)
            def _():
                _wait_den_scat(p)
            pltpu.make_async_copy(asrc_hbm.at[s_row], av_v.at[p],
                                  sema[p]).wait()
            pltpu.make_async_copy(adst_hbm.at[d_row], bv_v.at[p],
                                  semb[p]).wait()

            def g_body(g, _):
                e = (av_v[p, pl.ds(g * LANES, LANES)]
                     + bv_v[p, pl.ds(g * LANES, LANES)])
                e = jnp.where(e >= 0.0, e, 0.2 * e)
                ex = jnp.exp(e)
                gid = (base_edge + t * CHUNK + g * LANES
                       + lax.iota(jnp.int32, 16))
                ex = jnp.where(gid < n_edges, ex, 0.0)
                ex_v[p, pl.ds(g * LANES, LANES)] = ex
                return 0
            lax.fori_loop(0, CHUNK // LANES, g_body, 0)

            pltpu.async_copy(ex_v.at[p], den_sh.at[d_row], semd[p],
                             add=True)

            pltpu.make_async_copy(h_hbm.at[s_row], rows_v.at[p],
                                  semr[p]).wait()

            @pl.when(t > 0)
            def _():
                _wait_out_scat()

            def m_body(g, _):
                for j in range(LANES):
                    rr = g * LANES + j
                    sp = plsc.load_gather(
                        ex_v.at[p], [jnp.full((LANES,), rr, jnp.int32)])
                    for k in range(d_out // LANES):
                        srows_v[rr, pl.ds(k * LANES, LANES)] = (
                            rows_v[p, rr, pl.ds(k * LANES, LANES)] * sp)
                return 0
            lax.fori_loop(0, CHUNK // LANES, m_body, 0)

            pltpu.async_copy(srows_v, out_sh.at[d_row], semo, add=True)

            tn = t + 2

            @pl.when(tn < n_chunks)
            def _():
                @pl.when(tn % SUP == 0)
                def _():
                    _wait_idx_block(tn // SUP)
                _issue_gathers(tn, p)

        def _wait_idx_block(b):
            qb = b % 2
            pltpu.make_async_copy(ei_hbm.at[0, wid, b], sidx_v.at[qb],
                                  semidx).wait()
            pltpu.make_async_copy(ei_hbm.at[1, wid, b], didx_v.at[qb],
                                  semidx).wait()

        with jax.named_scope("sc_prologue"):
            pltpu.sync_copy(ei_hbm.at[0, wid, 0], sidx_v.at[0])
            pltpu.sync_copy(ei_hbm.at[1, wid, 0], didx_v.at[0])
            _issue_gathers(0, 0)
            _issue_gathers(1, 1)

        def pipe_body(u, _):
            ta = 2 * u
            @pl.when(ta % SUP == 0)
            def _():
                b = ta // SUP

                @pl.when(b + 1 < n_blocks)
                def _():
                    qn = (b + 1) % 2
                    pltpu.async_copy(ei_hbm.at[0, wid, b + 1], sidx_v.at[qn],
                                     semidx)
                    pltpu.async_copy(ei_hbm.at[1, wid, b + 1], didx_v.at[qn],
                                     semidx)

            _process(ta, 0)
            _process(ta + 1, 1)
            return 0

        with jax.named_scope("sc_mainloop"):
            lax.fori_loop(0, n_chunks // 2, pipe_body, 0)

        with jax.named_scope("sc_epilogue"):
            _wait_den_scat(0)
            _wait_den_scat(1)
            _wait_out_scat()

            plsc.subcore_barrier()

        with jax.named_scope("sc_drain"):
            @pl.when(sid < NS - 1)
            def _():
                pltpu.sync_copy(
                    out_sh.at[pl.ds(sid * row_blk, row_blk)],
                    outp_hbm.at[cid, pl.ds(sid * row_blk, row_blk)])

            @pl.when(sid == NS - 1)
            def _():
                pltpu.sync_copy(
                    out_sh.at[pl.ds((NS - 1) * row_blk, row_last)],
                    outp_hbm.at[cid, pl.ds((NS - 1) * row_blk, row_last)])

            @pl.when(sid == 0)
            def _():
                pltpu.sync_copy(den_sh,
                                denp_hbm.at[pl.ds(cid * n_pad, n_pad)])

    return sc_kernel


def kernel(input, edge_index, W, att_src, att_dst, bias):
    n, d_in = input.shape
    d_out = W.shape[1]
    n_edges = edge_index.shape[1]

    bn = 2000
    att2 = jnp.stack([att_src, att_dst], axis=1)
    h, a2 = pl.pallas_call(
        _proj_body,
        grid=(n // bn,),
        in_specs=[
            pl.BlockSpec((bn, d_in), lambda i: (i, 0)),
            pl.BlockSpec((d_in, d_out), lambda i: (0, 0)),
            pl.BlockSpec((d_out, 2), lambda i: (0, 0)),
        ],
        out_specs=[
            pl.BlockSpec((bn, d_out), lambda i: (i, 0)),
            pl.BlockSpec((bn, 2), lambda i: (i, 0)),
        ],
        out_shape=[
            jax.ShapeDtypeStruct((n, d_out), jnp.float32),
            jax.ShapeDtypeStruct((n, 2), jnp.float32),
        ],
    )(input, W, att2)
    asrc = a2[:, 0]
    adst = a2[:, 1]

    sup_per_tile = -(-n_edges // (NW * SUP * CHUNK))
    e_pad = NW * sup_per_tile * SUP * CHUNK
    npad_e = e_pad - n_edges
    pad_iota = lax.iota(jnp.int32, npad_e)
    pad_block = jnp.stack([pad_iota % n, n + (pad_iota % CHUNK)])
    ei = jnp.concatenate([edge_index.astype(jnp.int32), pad_block], axis=1)
    ei = ei.reshape(2, NW, sup_per_tile, SUP, CHUNK)

    n_pad = -(-n // 1024) * 1024
    sc = _make_sc_kernel(n, d_out, n_edges, sup_per_tile, n_pad)
    outp, denp = sc(h, asrc, adst, ei)

    denp3 = denp.reshape(NC, n_pad)[:, :n].reshape(NC, n, 1)
    bias2 = bias.reshape(1, d_out)
    out = pl.pallas_call(
        _final_body,
        grid=(n // bn,),
        in_specs=[
            pl.BlockSpec((NC, bn, d_out), lambda i: (0, i, 0)),
            pl.BlockSpec((NC, bn, 1), lambda i: (0, i, 0)),
            pl.BlockSpec((1, d_out), lambda i: (0, 0)),
        ],
        out_specs=pl.BlockSpec((bn, d_out), lambda i: (i, 0)),
        out_shape=jax.ShapeDtypeStruct((n, d_out), jnp.float32),
    )(outp, denp3, bias2)
    return out
